# Initial kernel scaffold; baseline (speedup 1.0000x reference)
import jax
import jax.numpy as jnp
from jax.experimental import pallas as pl


def kernel(grid_features, region_features, Wg, bg, Wr, br, centers, grid_bias, region_bias, r_bar, Wo, bo):
    raise NotImplementedError("write your pallas kernel here")


# speedup vs baseline: 1.9055x; 1.9055x over previous
import functools

import jax
import jax.numpy as jnp
from jax.experimental import pallas as pl
from jax.experimental.pallas import tpu as pltpu

B, M = 4, 65536
GD, H, K = 128, 256, 10
TM = 4096
MT = M // TM


def _agg_kernel(g_ref, wg_ref, bg_ref, cent_ref, rbar_ref, wo_ref, bo_ref,
                out_ref, acc_ref):
    m = pl.program_id(1)
    g = g_ref[0]
    v = jnp.dot(g, wg_ref[...], preferred_element_type=jnp.float32)
    v = v + bg_ref[...]
    dots = jax.lax.dot_general(v, cent_ref[...],
                               (((1,), (1,)), ((), ())),
                               preferred_element_type=jnp.float32)
    dots = dots - jnp.max(dots, axis=-1, keepdims=True)
    e = jnp.exp(dots)
    corr = e / jnp.sum(e, axis=-1, keepdims=True)
    gc = v - rbar_ref[...]
    part = jax.lax.dot_general(corr, gc,
                               (((0,), (0,)), ((), ())),
                               preferred_element_type=jnp.float32)

    @pl.when(m == 0)
    def _init():
        acc_ref[...] = part

    @pl.when(m > 0)
    def _accum():
        acc_ref[...] += part

    @pl.when(m == MT - 1)
    def _finalize():
        agg = acc_ref[...]
        norm = jnp.sqrt(jnp.sum(agg * agg, axis=-1, keepdims=True))
        agg = agg / jnp.maximum(norm, 1e-12)
        out_ref[0] = (jnp.dot(agg, wo_ref[...],
                              preferred_element_type=jnp.float32)
                      + bo_ref[...])


@functools.partial(jax.jit, static_argnames=())
def kernel(grid_features, region_features, Wg, bg, Wr, br, centers,
           grid_bias, region_bias, r_bar, Wo, bo):
    del region_features, Wr, br, grid_bias, region_bias
    bg2 = bg.reshape(1, H)
    rbar2 = r_bar.reshape(1, H)
    bo2 = bo.reshape(1, H)

    rep = lambda b, m: (0, 0)
    out = pl.pallas_call(
        _agg_kernel,
        grid=(B, MT),
        in_specs=[
            pl.BlockSpec((1, TM, GD), lambda b, m: (b, m, 0)),
            pl.BlockSpec((GD, H), rep),
            pl.BlockSpec((1, H), rep),
            pl.BlockSpec((K, H), rep),
            pl.BlockSpec((1, H), rep),
            pl.BlockSpec((H, H), rep),
            pl.BlockSpec((1, H), rep),
        ],
        out_specs=pl.BlockSpec((1, K, H), lambda b, m: (b, 0, 0)),
        out_shape=jax.ShapeDtypeStruct((B, K, H), jnp.float32),
        scratch_shapes=[pltpu.VMEM((K, H), jnp.float32)],
        compiler_params=pltpu.CompilerParams(
            dimension_semantics=("arbitrary", "arbitrary"),
        ),
    )(grid_features, Wg, bg2, centers, rbar2, Wo, bo2)
    return out


# speedup vs baseline: 2.3122x; 1.2134x over previous
import functools

import jax
import jax.numpy as jnp
from jax.experimental import pallas as pl
from jax.experimental.pallas import tpu as pltpu

B, M = 4, 65536
GD, H, K = 128, 256, 10
TM = 4096
MT = M // TM


def _agg_kernel(g_ref, wg_ref, bg_ref, cent_ref, rbar_ref, wo_ref, bo_ref,
                out_ref, a_ref, gacc_ref, sacc_ref):
    m = pl.program_id(1)

    @pl.when(m == 0)
    def _fold_weights():
        a_ref[...] = jax.lax.dot_general(
            wg_ref[...], cent_ref[...], (((1,), (1,)), ((), ())),
            preferred_element_type=jnp.float32)

    g = g_ref[0]
    dots = jnp.dot(g, a_ref[...], preferred_element_type=jnp.float32)
    dots = dots - jnp.max(dots, axis=-1, keepdims=True)
    e = jnp.exp(dots)
    corr = e / jnp.sum(e, axis=-1, keepdims=True)
    gpart = jax.lax.dot_general(corr, g, (((0,), (0,)), ((), ())),
                                preferred_element_type=jnp.float32)
    ones = jnp.ones((TM, 1), dtype=jnp.float32)
    spart = jax.lax.dot_general(corr, ones, (((0,), (0,)), ((), ())),
                                preferred_element_type=jnp.float32)

    @pl.when(m == 0)
    def _init():
        gacc_ref[...] = gpart
        sacc_ref[...] = spart

    @pl.when(m > 0)
    def _accum():
        gacc_ref[...] += gpart
        sacc_ref[...] += spart

    @pl.when(m == MT - 1)
    def _finalize():
        agg = jnp.dot(gacc_ref[...], wg_ref[...],
                      preferred_element_type=jnp.float32)
        agg = agg + sacc_ref[...] * (bg_ref[...] - rbar_ref[...])
        norm = jnp.sqrt(jnp.sum(agg * agg, axis=-1, keepdims=True))
        agg = agg / jnp.maximum(norm, 1e-12)
        out_ref[0] = (jnp.dot(agg, wo_ref[...],
                              preferred_element_type=jnp.float32)
                      + bo_ref[...])


@functools.partial(jax.jit, static_argnames=())
def kernel(grid_features, region_features, Wg, bg, Wr, br, centers,
           grid_bias, region_bias, r_bar, Wo, bo):
    del region_features, Wr, br, grid_bias, region_bias
    bg2 = bg.reshape(1, H)
    rbar2 = r_bar.reshape(1, H)
    bo2 = bo.reshape(1, H)

    rep = lambda b, m: (0, 0)
    out = pl.pallas_call(
        _agg_kernel,
        grid=(B, MT),
        in_specs=[
            pl.BlockSpec((1, TM, GD), lambda b, m: (b, m, 0)),
            pl.BlockSpec((GD, H), rep),
            pl.BlockSpec((1, H), rep),
            pl.BlockSpec((K, H), rep),
            pl.BlockSpec((1, H), rep),
            pl.BlockSpec((H, H), rep),
            pl.BlockSpec((1, H), rep),
        ],
        out_specs=pl.BlockSpec((1, K, H), lambda b, m: (b, 0, 0)),
        out_shape=jax.ShapeDtypeStruct((B, K, H), jnp.float32),
        scratch_shapes=[
            pltpu.VMEM((GD, K), jnp.float32),
            pltpu.VMEM((K, GD), jnp.float32),
            pltpu.VMEM((K, 1), jnp.float32),
        ],
        compiler_params=pltpu.CompilerParams(
            dimension_semantics=("arbitrary", "arbitrary"),
        ),
    )(grid_features, Wg, bg2, centers, rbar2, Wo, bo2)
    return out


# speedup vs baseline: 4.0842x; 1.7664x over previous
import functools

import jax
import jax.numpy as jnp
from jax.experimental import pallas as pl
from jax.experimental.pallas import tpu as pltpu

B, M = 4, 65536
GD, H, K = 128, 256, 10
TM = 4096
MT = M // TM


def _agg_kernel(g_ref, wg_ref, bg_ref, cent_ref, rbar_ref, wo_ref, bo_ref,
                out_ref, a_ref, gacc_ref, sacc_ref):
    m = pl.program_id(1)

    @pl.when(m == 0)
    def _fold_weights():
        a_ref[...] = jax.lax.dot_general(
            wg_ref[...], cent_ref[...], (((1,), (1,)), ((), ())),
            preferred_element_type=jnp.float32)

    g = g_ref[0]
    dots = jax.lax.dot_general(a_ref[...], g, (((0,), (1,)), ((), ())),
                               preferred_element_type=jnp.float32)
    e = jnp.exp(dots)
    corr = e / jnp.sum(e, axis=0, keepdims=True)
    gpart = jax.lax.dot_general(corr, g, (((1,), (0,)), ((), ())),
                                preferred_element_type=jnp.float32)
    spart = jnp.sum(corr, axis=1, keepdims=True)

    @pl.when(m == 0)
    def _init():
        gacc_ref[...] = gpart
        sacc_ref[...] = spart

    @pl.when(m > 0)
    def _accum():
        gacc_ref[...] += gpart
        sacc_ref[...] += spart

    @pl.when(m == MT - 1)
    def _finalize():
        agg = jnp.dot(gacc_ref[...], wg_ref[...],
                      preferred_element_type=jnp.float32)
        agg = agg + sacc_ref[...] * (bg_ref[...] - rbar_ref[...])
        norm = jnp.sqrt(jnp.sum(agg * agg, axis=-1, keepdims=True))
        agg = agg / jnp.maximum(norm, 1e-12)
        out_ref[0] = (jnp.dot(agg, wo_ref[...],
                              preferred_element_type=jnp.float32)
                      + bo_ref[...])


@functools.partial(jax.jit, static_argnames=())
def kernel(grid_features, region_features, Wg, bg, Wr, br, centers,
           grid_bias, region_bias, r_bar, Wo, bo):
    del region_features, Wr, br, grid_bias, region_bias
    bg2 = bg.reshape(1, H)
    rbar2 = r_bar.reshape(1, H)
    bo2 = bo.reshape(1, H)

    rep = lambda b, m: (0, 0)
    out = pl.pallas_call(
        _agg_kernel,
        grid=(B, MT),
        in_specs=[
            pl.BlockSpec((1, TM, GD), lambda b, m: (b, m, 0)),
            pl.BlockSpec((GD, H), rep),
            pl.BlockSpec((1, H), rep),
            pl.BlockSpec((K, H), rep),
            pl.BlockSpec((1, H), rep),
            pl.BlockSpec((H, H), rep),
            pl.BlockSpec((1, H), rep),
        ],
        out_specs=pl.BlockSpec((1, K, H), lambda b, m: (b, 0, 0)),
        out_shape=jax.ShapeDtypeStruct((B, K, H), jnp.float32),
        scratch_shapes=[
            pltpu.VMEM((GD, K), jnp.float32),
            pltpu.VMEM((K, GD), jnp.float32),
            pltpu.VMEM((K, 1), jnp.float32),
        ],
        compiler_params=pltpu.CompilerParams(
            dimension_semantics=("arbitrary", "arbitrary"),
        ),
    )(grid_features, Wg, bg2, centers, rbar2, Wo, bo2)
    return out


# speedup vs baseline: 6.7629x; 1.6559x over previous
import functools

import jax
import jax.numpy as jnp
from jax.experimental import pallas as pl
from jax.experimental.pallas import tpu as pltpu

B, M = 4, 65536
GD, H, K = 128, 256, 10
TM = 32768
MT = M // TM


def _agg_kernel(g_ref, wg_ref, bg_ref, cent_ref, rbar_ref, wo_ref, bo_ref,
                out_ref, a_ref, gacc_ref, sacc_ref):
    m = pl.program_id(1)

    @pl.when(m == 0)
    def _fold_weights():
        a_ref[...] = jax.lax.dot_general(
            wg_ref[...], cent_ref[...], (((1,), (1,)), ((), ())),
            preferred_element_type=jnp.float32)

    g = g_ref[0]
    dots = jax.lax.dot_general(a_ref[...], g, (((0,), (1,)), ((), ())),
                               preferred_element_type=jnp.float32)
    e = jnp.exp(dots)
    corr = e / jnp.sum(e, axis=0, keepdims=True)
    gpart = jax.lax.dot_general(corr, g, (((1,), (0,)), ((), ())),
                                preferred_element_type=jnp.float32)
    spart = jnp.sum(corr, axis=1, keepdims=True)

    @pl.when(m == 0)
    def _init():
        gacc_ref[...] = gpart
        sacc_ref[...] = spart

    @pl.when(m > 0)
    def _accum():
        gacc_ref[...] += gpart
        sacc_ref[...] += spart

    @pl.when(m == MT - 1)
    def _finalize():
        agg = jnp.dot(gacc_ref[...], wg_ref[...],
                      preferred_element_type=jnp.float32)
        agg = agg + sacc_ref[...] * (bg_ref[...] - rbar_ref[...])
        norm = jnp.sqrt(jnp.sum(agg * agg, axis=-1, keepdims=True))
        agg = agg / jnp.maximum(norm, 1e-12)
        out_ref[0] = (jnp.dot(agg, wo_ref[...],
                              preferred_element_type=jnp.float32)
                      + bo_ref[...])


@functools.partial(jax.jit, static_argnames=())
def kernel(grid_features, region_features, Wg, bg, Wr, br, centers,
           grid_bias, region_bias, r_bar, Wo, bo):
    del region_features, Wr, br, grid_bias, region_bias
    bg2 = bg.reshape(1, H)
    rbar2 = r_bar.reshape(1, H)
    bo2 = bo.reshape(1, H)

    rep = lambda b, m: (0, 0)
    out = pl.pallas_call(
        _agg_kernel,
        grid=(B, MT),
        in_specs=[
            pl.BlockSpec((1, TM, GD), lambda b, m: (b, m, 0)),
            pl.BlockSpec((GD, H), rep),
            pl.BlockSpec((1, H), rep),
            pl.BlockSpec((K, H), rep),
            pl.BlockSpec((1, H), rep),
            pl.BlockSpec((H, H), rep),
            pl.BlockSpec((1, H), rep),
        ],
        out_specs=pl.BlockSpec((1, K, H), lambda b, m: (b, 0, 0)),
        out_shape=jax.ShapeDtypeStruct((B, K, H), jnp.float32),
        scratch_shapes=[
            pltpu.VMEM((GD, K), jnp.float32),
            pltpu.VMEM((K, GD), jnp.float32),
            pltpu.VMEM((K, 1), jnp.float32),
        ],
        compiler_params=pltpu.CompilerParams(
            dimension_semantics=("arbitrary", "arbitrary"),
        ),
    )(grid_features, Wg, bg2, centers, rbar2, Wo, bo2)
    return out
